# baseline (speedup 1.0000x reference)
import functools

import numpy as np
import jax
import jax.numpy as jnp
from jax import lax
from jax.experimental import pallas as pl
from jax.experimental.pallas import tpu as pltpu
from jax.experimental.pallas import tpu_sc as plsc

H_IMG = 256
W_IMG = 256
N_PTS = 512
HROWS = H_IMG // 2
GROW = 7
PAD_C = 2
MCOLS = W_IMG + 2 * PAD_C
MROWS = HROWS + 2 * GROW
MWORDS = MROWS * MCOLS
MALLOC = ((MWORDS + 15) // 16) * 16
RL_LO = 3 - GROW
RL_HI = HROWS - 4 + GROW

_dyg, _dxg = np.meshgrid(np.arange(-2, 3), np.arange(-2, 3), indexing="ij")
_dyv = _dyg.ravel().astype(np.int64)
_dxv = _dxg.ravel().astype(np.int64)
_wv = np.exp(-np.sqrt(_dxv * _dxv + _dyv * _dyv) / 2.0).astype(np.float32)
_dy2 = np.concatenate([_dyv[16:], np.array([-3, -3, -3, -3, -3, 3, 3])])
_dx2 = np.concatenate([_dxv[16:], np.array([-2, -1, 0, 1, 2, 1, 2])])
_OFF1 = (_dyv[:16] * MCOLS + _dxv[:16]).astype(np.int32)
_OFF2 = (_dy2 * MCOLS + _dx2).astype(np.int32)
_W1 = _wv[:16].copy()
_W2 = np.concatenate([_wv[16:], np.zeros(7, np.float32)]).astype(np.float32)

_GDN = lax.GatherDimensionNumbers(
    offset_dims=(), collapsed_slice_dims=(0,), start_index_map=(0,))


def _sc_body(pred_hbm, pts_hbm, ca_hbm, offs_hbm, wts_hbm, out_hbm,
             map_v, pbuf_v, ptsv, bv, cav, offv, wtv, res_v, psem):
    w = lax.axis_index("s") * 2 + lax.axis_index("c")
    i = w >> 1
    h = w & 1

    pred_cp = pltpu.async_copy(
        pred_hbm.at[i, 0, pl.ds(h * HROWS, HROWS)], pbuf_v, psem)
    pltpu.sync_copy(pts_hbm.at[i], ptsv)
    pltpu.sync_copy(ca_hbm, cav)
    pltpu.sync_copy(offs_hbm, offv)
    pltpu.sync_copy(wts_hbm, wtv)

    zf = jnp.zeros((16,), jnp.float32)

    def zero_body(z, _):
        map_v[pl.ds(z * 16, 16)] = zf
        return 0
    lax.fori_loop(0, MALLOC // 16, zero_body, 0, unroll=8)

    ca = cav[...]
    lane = lax.iota(jnp.int32, 16)
    col0 = jnp.zeros((16,), jnp.int32)
    col1 = jnp.full((16,), 1, jnp.int32)
    lo = h * HROWS

    def coord_body(g, _):
        rows = g * 16 + lane
        px = plsc.load_gather(ptsv, [rows, col0]).astype(jnp.float32)
        py = plsc.load_gather(ptsv, [rows, col1]).astype(jnp.float32)
        cx = jnp.clip(px / ca, 0.0, float(W_IMG - 1)).astype(jnp.int32)
        cy = jnp.clip(py / ca, 0.0, float(H_IMG - 1)).astype(jnp.int32)
        rl = jnp.clip(cy - lo, RL_LO, RL_HI)
        bv[pl.ds(g * 16, 16)] = (rl + GROW) * MCOLS + (cx + PAD_C)
        return 0
    lax.fori_loop(0, N_PTS // 16, coord_body, 0)

    off1 = offv[pl.ds(0, 16)]
    off2 = offv[pl.ds(16, 16)]
    w1 = wtv[pl.ds(0, 16)]
    w2 = wtv[pl.ds(16, 16)]
    lane_ids = [jnp.full((16, 1), l, jnp.int32) for l in range(16)]

    def scat_body(g, _):
        b16 = bv[pl.ds(g * 16, 16)]
        for l in range(16):
            bb = lax.gather(b16, lane_ids[l], _GDN, (1,),
                            mode=lax.GatherScatterMode.PROMISE_IN_BOUNDS)
            plsc.addupdate_scatter(map_v, [bb + off1], w1)
            plsc.addupdate_scatter(map_v, [bb + off2], w2)
        return 0
    lax.fori_loop(0, N_PTS // 16, scat_body, 0)

    pred_cp.wait()

    row0 = GROW * MCOLS + PAD_C

    def red_body(r, acc):
        a_s, a_q, a_c, a_t, a_2 = acc
        mb = row0 + r * MCOLS
        for k in range(W_IMG // 16):
            p = pbuf_v[r, pl.ds(k * 16, 16)]
            t = map_v[pl.ds(mb + k * 16, 16)]
            a_s = a_s + p
            a_q = a_q + p * p
            a_c = a_c + p * t
            a_t = a_t + t
            a_2 = a_2 + t * t
        return (a_s, a_q, a_c, a_t, a_2)

    accs = lax.fori_loop(0, HROWS, red_body, (zf, zf, zf, zf, zf))
    for ridx in range(5):
        res_v[pl.ds(ridx * 16, 16)] = accs[ridx]
    pltpu.sync_copy(res_v, out_hbm.at[w])


@functools.lru_cache(maxsize=1)
def _sc_call():
    mesh = plsc.VectorSubcoreMesh(core_axis_name="c", subcore_axis_name="s")
    return pl.kernel(
        _sc_body,
        out_type=jax.ShapeDtypeStruct((32, 80), jnp.float32),
        mesh=mesh,
        scratch_types=[
            pltpu.VMEM((MALLOC,), jnp.float32),
            pltpu.VMEM((HROWS, W_IMG), jnp.float32),
            pltpu.VMEM((N_PTS, 2), jnp.int32),
            pltpu.VMEM((N_PTS,), jnp.int32),
            pltpu.VMEM((16,), jnp.float32),
            pltpu.VMEM((32,), jnp.int32),
            pltpu.VMEM((32,), jnp.float32),
            pltpu.VMEM((80,), jnp.float32),
            pltpu.SemaphoreType.DMA,
        ],
        compiler_params=pltpu.CompilerParams(
            needs_layout_passes=False, use_tc_tiling_on_sc=False),
    )


def kernel(pred, points_list, cell_area, log_scale):
    B = pred.shape[0]
    H, W = pred.shape[-2], pred.shape[-1]
    N = points_list.shape[1]
    ca16 = jnp.full((16,), cell_area, jnp.float32)
    offs = jnp.concatenate([jnp.asarray(_OFF1), jnp.asarray(_OFF2)])
    wts = jnp.concatenate([jnp.asarray(_W1), jnp.asarray(_W2)])

    out = _sc_call()(pred, points_list, ca16, offs, wts)
    part = out.reshape(B, 2, 5, 16).sum(axis=(1, 3))
    s_p, s_q, s_c, s_t, s_t2 = (part[:, r] for r in range(5))

    a = s_p + 1e-8
    ts = jnp.where(s_t > 0, s_t, 1.0)
    sp = (s_q / (a * a) - 2.0 * s_c / (a * ts) + s_t2 / (ts * ts)) / (H * W)
    sp = jnp.where(s_t > 0, sp, 0.0)
    spatial_loss = jnp.mean(sp)
    count_loss = jnp.mean(jnp.abs(s_p / cell_area - float(N)))
    scale = jnp.exp(log_scale)
    scale_loss = jnp.mean(jnp.maximum(8.0 - scale, 0.0)
                          + jnp.maximum(scale - 64.0, 0.0))
    total = 2.0 * count_loss + 0.15 * spatial_loss + 0.5 * scale_loss
    return total, count_loss, spatial_loss, scale_loss


# speedup vs baseline: 1.2618x; 1.2618x over previous
import functools

import numpy as np
import jax
import jax.numpy as jnp
from jax import lax
from jax.experimental import pallas as pl
from jax.experimental.pallas import tpu as pltpu
from jax.experimental.pallas import tpu_sc as plsc

H_IMG = 256
W_IMG = 256
N_PTS = 512
HROWS = H_IMG // 2
GROW = 7
PAD_C = 2
MCOLS = W_IMG + 2 * PAD_C
MROWS = HROWS + 2 * GROW
MWORDS = MROWS * MCOLS
MALLOC = ((MWORDS + 15) // 16) * 16
RL_LO = 3 - GROW
RL_HI = HROWS - 4 + GROW

_dyg, _dxg = np.meshgrid(np.arange(-2, 3), np.arange(-2, 3), indexing="ij")
_dyv = _dyg.ravel().astype(np.int64)
_dxv = _dxg.ravel().astype(np.int64)
_wv = np.exp(-np.sqrt(_dxv * _dxv + _dyv * _dyv) / 2.0).astype(np.float32)
_dy2 = np.concatenate([_dyv[16:], np.array([-3, -3, -3, -3, -3, 3, 3])])
_dx2 = np.concatenate([_dxv[16:], np.array([-2, -1, 0, 1, 2, 1, 2])])
_OFF1 = (_dyv[:16] * MCOLS + _dxv[:16]).astype(np.int32)
_OFF2 = (_dy2 * MCOLS + _dx2).astype(np.int32)
_W1 = _wv[:16].copy()
_W2 = np.concatenate([_wv[16:], np.zeros(7, np.float32)]).astype(np.float32)

_GDN = lax.GatherDimensionNumbers(
    offset_dims=(), collapsed_slice_dims=(0,), start_index_map=(0,))


def _sc_body(pred_hbm, pts_hbm, ca_hbm, offs_hbm, wts_hbm, out_hbm,
             map_v, pbuf_v, ptsv, bv, cav, offv, wtv, res_v, psem):
    w = lax.axis_index("s") * 2 + lax.axis_index("c")
    i = w >> 1
    h = w & 1

    pred_cp = pltpu.async_copy(
        pred_hbm.at[i, 0, pl.ds(h * HROWS, HROWS)], pbuf_v, psem)
    pltpu.sync_copy(pts_hbm.at[i], ptsv)
    pltpu.sync_copy(ca_hbm, cav)
    pltpu.sync_copy(offs_hbm, offv)
    pltpu.sync_copy(wts_hbm, wtv)

    zf = jnp.zeros((16,), jnp.float32)

    def zero_body(z, _):
        map_v[pl.ds(z * 16, 16)] = zf
        return 0
    lax.fori_loop(0, MALLOC // 16, zero_body, 0, unroll=8)

    ca = cav[...]
    lane2 = lax.iota(jnp.int32, 16) * 2
    lo = h * HROWS

    def coord_body(g, _):
        ev = g * 32 + lane2
        px = plsc.load_gather(ptsv, [ev]).astype(jnp.float32)
        py = plsc.load_gather(ptsv, [ev + 1]).astype(jnp.float32)
        cx = jnp.clip(px / ca, 0.0, float(W_IMG - 1)).astype(jnp.int32)
        cy = jnp.clip(py / ca, 0.0, float(H_IMG - 1)).astype(jnp.int32)
        rl = jnp.clip(cy - lo, RL_LO, RL_HI)
        bv[pl.ds(g * 16, 16)] = (rl + GROW) * MCOLS + (cx + PAD_C)
        return 0
    lax.fori_loop(0, N_PTS // 16, coord_body, 0)

    off1 = offv[pl.ds(0, 16)]
    off2 = offv[pl.ds(16, 16)]
    w1 = wtv[pl.ds(0, 16)]
    w2 = wtv[pl.ds(16, 16)]
    lane_ids = [jnp.full((16, 1), l, jnp.int32) for l in range(16)]

    def scat_body(g, _):
        b16 = bv[pl.ds(g * 16, 16)]
        for l in range(16):
            bb = lax.gather(b16, lane_ids[l], _GDN, (1,),
                            mode=lax.GatherScatterMode.PROMISE_IN_BOUNDS)
            plsc.addupdate_scatter(map_v, [bb + off1], w1)
            plsc.addupdate_scatter(map_v, [bb + off2], w2)
        return 0
    lax.fori_loop(0, N_PTS // 16, scat_body, 0)

    pred_cp.wait()

    row0 = GROW * MCOLS + PAD_C

    def red_body(r, acc):
        a_s, a_q, a_c, a_t, a_2 = acc
        mb = row0 + r * MCOLS
        for k in range(W_IMG // 16):
            p = pbuf_v[r, pl.ds(k * 16, 16)]
            t = map_v[pl.ds(mb + k * 16, 16)]
            a_s = a_s + p
            a_q = a_q + p * p
            a_c = a_c + p * t
            a_t = a_t + t
            a_2 = a_2 + t * t
        return (a_s, a_q, a_c, a_t, a_2)

    accs = lax.fori_loop(0, HROWS, red_body, (zf, zf, zf, zf, zf))
    for ridx in range(5):
        res_v[pl.ds(ridx * 16, 16)] = accs[ridx]
    pltpu.sync_copy(res_v, out_hbm.at[w])


@functools.lru_cache(maxsize=1)
def _sc_call():
    mesh = plsc.VectorSubcoreMesh(core_axis_name="c", subcore_axis_name="s")
    return pl.kernel(
        _sc_body,
        out_type=jax.ShapeDtypeStruct((32, 80), jnp.float32),
        mesh=mesh,
        scratch_types=[
            pltpu.VMEM((MALLOC,), jnp.float32),
            pltpu.VMEM((HROWS, W_IMG), jnp.float32),
            pltpu.VMEM((2 * N_PTS,), jnp.int32),
            pltpu.VMEM((N_PTS,), jnp.int32),
            pltpu.VMEM((16,), jnp.float32),
            pltpu.VMEM((32,), jnp.int32),
            pltpu.VMEM((32,), jnp.float32),
            pltpu.VMEM((80,), jnp.float32),
            pltpu.SemaphoreType.DMA,
        ],
        compiler_params=pltpu.CompilerParams(needs_layout_passes=False),
    )


def kernel(pred, points_list, cell_area, log_scale):
    B = pred.shape[0]
    H, W = pred.shape[-2], pred.shape[-1]
    N = points_list.shape[1]
    ca16 = jnp.full((16,), cell_area, jnp.float32)
    offs = jnp.concatenate([jnp.asarray(_OFF1), jnp.asarray(_OFF2)])
    wts = jnp.concatenate([jnp.asarray(_W1), jnp.asarray(_W2)])
    pts2 = points_list.reshape(B, 2 * N)

    out = _sc_call()(pred, pts2, ca16, offs, wts)
    part = out.reshape(B, 2, 5, 16).sum(axis=(1, 3))
    s_p, s_q, s_c, s_t, s_t2 = (part[:, r] for r in range(5))

    a = s_p + 1e-8
    ts = jnp.where(s_t > 0, s_t, 1.0)
    sp = (s_q / (a * a) - 2.0 * s_c / (a * ts) + s_t2 / (ts * ts)) / (H * W)
    sp = jnp.where(s_t > 0, sp, 0.0)
    spatial_loss = jnp.mean(sp)
    count_loss = jnp.mean(jnp.abs(s_p / cell_area - float(N)))
    scale = jnp.exp(log_scale)
    scale_loss = jnp.mean(jnp.maximum(8.0 - scale, 0.0)
                          + jnp.maximum(scale - 64.0, 0.0))
    total = 2.0 * count_loss + 0.15 * spatial_loss + 0.5 * scale_loss
    return total, count_loss, spatial_loss, scale_loss
